# baseline (speedup 1.0000x reference)
import functools

import jax
import jax.numpy as jnp
from jax import lax
from jax.experimental import pallas as pl
from jax.experimental.pallas import tpu as pltpu
from jax.experimental.pallas import tpu_sc as plsc

VOCAB = 1000000
EMBED_DIM = 64
BATCH = 4096
SEQ = 200

_INFO = plsc.get_sparse_core_info()
_NC, _NS = _INFO.num_cores, _INFO.num_subcores
_NW = _NC * _NS
_ROWS = SEQ * BATCH
_IW = 128
_PD = 2 * EMBED_DIM
_L = 16
_S = 4
_LA = 2
_NBLK = SEQ // _S


def _sc_gather(table, x128):
    mesh = plsc.VectorSubcoreMesh(core_axis_name="c", subcore_axis_name="s")

    @functools.partial(
        pl.kernel,
        mesh=mesh,
        out_type=jax.ShapeDtypeStruct((SEQ, BATCH, EMBED_DIM), jnp.float32),
        scratch_types=[
            pltpu.VMEM((SEQ, _IW), jnp.int32),
            pltpu.VMEM((_S, _IW), jnp.int32),
            pltpu.VMEM((_S, _IW, EMBED_DIM), jnp.float32),
            pltpu.SemaphoreType.DMA((_S,)),
            pltpu.SemaphoreType.DMA((_S,)),
        ],
        compiler_params=pltpu.CompilerParams(
            use_tc_tiling_on_sc=False, needs_layout_passes=False),
    )
    def k(table_hbm, x_hbm, out_hbm, xblk_v, idx_v, rows_v, sem_g, sem_w):
        wid = lax.axis_index("s") * _NC + lax.axis_index("c")
        b0 = wid * _IW
        pltpu.sync_copy(x_hbm.at[pl.ds(wid * SEQ, SEQ), :], xblk_v)

        def build_idx(s, slot):
            for v in range(_IW // _L):
                flat = (lax.iota(jnp.int32, _L) + (v * _L)) * SEQ + s
                rows = lax.shift_right_logical(flat, 7)
                cols = lax.bitwise_and(flat, 127)
                g = plsc.load_gather(xblk_v, [rows, cols])
                idx_v[slot, pl.ds(v * _L, _L)] = g

        def fire_g(slot):
            return pltpu.async_copy(
                table_hbm.at[idx_v.at[slot]], rows_v.at[slot],
                sem_g.at[slot])

        def fire_w(s, slot):
            return pltpu.async_copy(
                rows_v.at[slot],
                out_hbm.at[s, pl.ds(b0, _IW), :],
                sem_w.at[slot])

        def wait_g(slot):
            pltpu.make_async_copy(
                table_hbm.at[idx_v.at[0]], rows_v.at[slot],
                sem_g.at[slot]).wait()

        def wait_w(slot):
            pltpu.make_async_copy(
                rows_v.at[slot],
                out_hbm.at[0, pl.ds(0, _IW), :],
                sem_w.at[slot]).wait()

        for s in range(_LA):
            build_idx(s, s % _S)
            fire_g(s % _S)

        for u in range(_S):
            s = u
            if s - (_S - _LA) >= 0:
                wait_w((s + _LA) % _S)
            build_idx(s + _LA, (s + _LA) % _S)
            fire_g((s + _LA) % _S)
            wait_g(u)
            fire_w(s, u)

        def block(blk, carry):
            for u in range(_S):
                s = blk * _S + u
                wait_w((u + _LA) % _S)
                build_idx(s + _LA, (u + _LA) % _S)
                fire_g((u + _LA) % _S)
                wait_g(u)
                fire_w(s, u)
            return carry

        lax.fori_loop(1, _NBLK - 1, block, 0)

        for u in range(_S):
            s = (_NBLK - 1) * _S + u
            wait_w((u + _LA) % _S)
            if s + _LA < SEQ:
                build_idx(s + _LA, (u + _LA) % _S)
                fire_g((u + _LA) % _S)
            wait_g(u)
            fire_w(s, u)

        for u in range(_S - _LA, _S):
            wait_w(u)

    return k(table, x128)


def kernel(x, table):
    x128 = x.reshape(BATCH * SEQ // _IW, _IW)
    return _sc_gather(table, x128)


# speedup vs baseline: 1.3366x; 1.3366x over previous
import functools

import jax
import jax.numpy as jnp
from jax import lax
from jax.experimental import pallas as pl
from jax.experimental.pallas import tpu as pltpu
from jax.experimental.pallas import tpu_sc as plsc

VOCAB = 1000000
EMBED_DIM = 64
BATCH = 4096
SEQ = 200

_INFO = plsc.get_sparse_core_info()
_NC, _NS = _INFO.num_cores, _INFO.num_subcores
_NW = _NC * _NS
_ROWS = SEQ * BATCH
_IW = 128
_PD = 2 * EMBED_DIM
_L = 16
_S = 4
_LA = 2
_NBLK = SEQ // _S


def _sc_gather(table, x128):
    mesh = plsc.VectorSubcoreMesh(core_axis_name="c", subcore_axis_name="s")

    @functools.partial(
        pl.kernel,
        mesh=mesh,
        out_type=jax.ShapeDtypeStruct((SEQ, BATCH, _PD), jnp.float32),
        scratch_types=[
            pltpu.VMEM((SEQ, _IW), jnp.int32),
            pltpu.VMEM((_S, _IW), jnp.int32),
            pltpu.VMEM((_S, _IW, EMBED_DIM), jnp.float32),
            pltpu.SemaphoreType.DMA((_S,)),
            pltpu.SemaphoreType.DMA((_S,)),
        ],
        compiler_params=pltpu.CompilerParams(
            use_tc_tiling_on_sc=False, needs_layout_passes=False),
    )
    def k(table_hbm, x_hbm, out_hbm, xblk_v, idx_v, rows_v, sem_g, sem_w):
        wid = lax.axis_index("s") * _NC + lax.axis_index("c")
        b0 = wid * _IW
        pltpu.sync_copy(x_hbm.at[pl.ds(wid * SEQ, SEQ), :], xblk_v)

        def build_idx(s, slot):
            for v in range(_IW // _L):
                flat = (lax.iota(jnp.int32, _L) + (v * _L)) * SEQ + s
                rows = lax.shift_right_logical(flat, 7)
                cols = lax.bitwise_and(flat, 127)
                g = plsc.load_gather(xblk_v, [rows, cols])
                idx_v[slot, pl.ds(v * _L, _L)] = g

        def fire_g(slot):
            return pltpu.async_copy(
                table_hbm.at[idx_v.at[slot]], rows_v.at[slot],
                sem_g.at[slot])

        def fire_w(s, slot):
            return pltpu.async_copy(
                rows_v.at[slot],
                out_hbm.at[s, pl.ds(b0, _IW), pl.ds(0, EMBED_DIM)],
                sem_w.at[slot])

        def wait_g(slot):
            pltpu.make_async_copy(
                table_hbm.at[idx_v.at[0]], rows_v.at[slot],
                sem_g.at[slot]).wait()

        def wait_w(slot):
            pltpu.make_async_copy(
                rows_v.at[slot],
                out_hbm.at[0, pl.ds(0, _IW), pl.ds(0, EMBED_DIM)],
                sem_w.at[slot]).wait()

        for s in range(_LA):
            build_idx(s, s % _S)
            fire_g(s % _S)

        for u in range(_S):
            s = u
            if s - (_S - _LA) >= 0:
                wait_w((s + _LA) % _S)
            build_idx(s + _LA, (s + _LA) % _S)
            fire_g((s + _LA) % _S)
            wait_g(u)
            fire_w(s, u)

        def block(blk, carry):
            for u in range(_S):
                s = blk * _S + u
                wait_w((u + _LA) % _S)
                build_idx(s + _LA, (u + _LA) % _S)
                fire_g((u + _LA) % _S)
                wait_g(u)
                fire_w(s, u)
            return carry

        lax.fori_loop(1, _NBLK - 1, block, 0)

        for u in range(_S):
            s = (_NBLK - 1) * _S + u
            wait_w((u + _LA) % _S)
            if s + _LA < SEQ:
                build_idx(s + _LA, (u + _LA) % _S)
                fire_g((u + _LA) % _S)
            wait_g(u)
            fire_w(s, u)

        for u in range(_S - _LA, _S):
            wait_w(u)

    return k(table, x128)


def kernel(x, table):
    x128 = x.reshape(BATCH * SEQ // _IW, _IW)
    return _sc_gather(table, x128)[..., :EMBED_DIM]


# speedup vs baseline: 1.3376x; 1.0008x over previous
import functools

import jax
import jax.numpy as jnp
from jax import lax
from jax.experimental import pallas as pl
from jax.experimental.pallas import tpu as pltpu
from jax.experimental.pallas import tpu_sc as plsc

VOCAB = 1000000
EMBED_DIM = 64
BATCH = 4096
SEQ = 200

_INFO = plsc.get_sparse_core_info()
_NC, _NS = _INFO.num_cores, _INFO.num_subcores
_NW = _NC * _NS
_ROWS = SEQ * BATCH
_IW = 128
_PD = 2 * EMBED_DIM
_L = 16
_S = 8
_LA = 5
_NBLK = SEQ // _S


def _sc_gather(table, x128):
    mesh = plsc.VectorSubcoreMesh(core_axis_name="c", subcore_axis_name="s")

    @functools.partial(
        pl.kernel,
        mesh=mesh,
        out_type=jax.ShapeDtypeStruct((SEQ, BATCH, _PD), jnp.float32),
        scratch_types=[
            pltpu.VMEM((SEQ, _IW), jnp.int32),
            pltpu.VMEM((_S, _IW), jnp.int32),
            pltpu.VMEM((_S, _IW, EMBED_DIM), jnp.float32),
            pltpu.SemaphoreType.DMA((_S,)),
            pltpu.SemaphoreType.DMA((_S,)),
        ],
        compiler_params=pltpu.CompilerParams(
            use_tc_tiling_on_sc=False, needs_layout_passes=False),
    )
    def k(table_hbm, x_hbm, out_hbm, xblk_v, idx_v, rows_v, sem_g, sem_w):
        wid = lax.axis_index("s") * _NC + lax.axis_index("c")
        b0 = wid * _IW
        pltpu.sync_copy(x_hbm.at[pl.ds(wid * SEQ, SEQ), :], xblk_v)

        def build_idx(s, slot):
            for v in range(_IW // _L):
                flat = (lax.iota(jnp.int32, _L) + (v * _L)) * SEQ + s
                rows = lax.shift_right_logical(flat, 7)
                cols = lax.bitwise_and(flat, 127)
                g = plsc.load_gather(xblk_v, [rows, cols])
                idx_v[slot, pl.ds(v * _L, _L)] = g

        def fire_g(slot):
            return pltpu.async_copy(
                table_hbm.at[idx_v.at[slot]], rows_v.at[slot],
                sem_g.at[slot])

        def fire_w(s, slot):
            return pltpu.async_copy(
                rows_v.at[slot],
                out_hbm.at[s, pl.ds(b0, _IW), pl.ds(0, EMBED_DIM)],
                sem_w.at[slot])

        def wait_g(slot):
            pltpu.make_async_copy(
                table_hbm.at[idx_v.at[0]], rows_v.at[slot],
                sem_g.at[slot]).wait()

        def wait_w(slot):
            pltpu.make_async_copy(
                rows_v.at[slot],
                out_hbm.at[0, pl.ds(0, _IW), pl.ds(0, EMBED_DIM)],
                sem_w.at[slot]).wait()

        for s in range(_LA):
            build_idx(s, s % _S)
            fire_g(s % _S)

        for u in range(_S):
            s = u
            if s - (_S - _LA) >= 0:
                wait_w((s + _LA) % _S)
            build_idx(s + _LA, (s + _LA) % _S)
            fire_g((s + _LA) % _S)
            wait_g(u)
            fire_w(s, u)

        def block(blk, carry):
            for u in range(_S):
                s = blk * _S + u
                wait_w((u + _LA) % _S)
                build_idx(s + _LA, (u + _LA) % _S)
                fire_g((u + _LA) % _S)
                wait_g(u)
                fire_w(s, u)
            return carry

        lax.fori_loop(1, _NBLK - 1, block, 0)

        for u in range(_S):
            s = (_NBLK - 1) * _S + u
            wait_w((u + _LA) % _S)
            if s + _LA < SEQ:
                build_idx(s + _LA, (u + _LA) % _S)
                fire_g((u + _LA) % _S)
            wait_g(u)
            fire_w(s, u)

        for j in range(SEQ - (_S - _LA), SEQ):
            wait_w(j % _S)

    return k(table, x128)


def kernel(x, table):
    x128 = x.reshape(BATCH * SEQ // _IW, _IW)
    return _sc_gather(table, x128)[..., :EMBED_DIM]
